# Initial kernel scaffold; baseline (speedup 1.0000x reference)
import jax
import jax.numpy as jnp
from jax.experimental import pallas as pl


def kernel(query_embeddings, doc_embeddings, neg_doc_embeddings):
    raise NotImplementedError("write your pallas kernel here")


# speedup vs baseline: 7.4956x; 7.4956x over previous
import functools

import jax
import jax.numpy as jnp
from jax.experimental import pallas as pl
from jax.experimental.pallas import tpu as pltpu

_B, _NQ, _NS, _D = 16, 512, 2048, 64
_K = 128


def _score_kernel(q_ref, d_ref, out_ref, kscratch):
    qb = q_ref[0]
    db = d_ref[0]
    s = jax.lax.dot_general(
        qb, db, (((1,), (1,)), ((), ())),
        preferred_element_type=jnp.float32)

    rowmax = jnp.max(s, axis=1, keepdims=True)
    bits = jax.lax.bitcast_convert_type(s, jnp.int32)
    keys = jnp.where(bits < 0, bits ^ jnp.int32(0x7FFFFFFF), bits)
    kscratch[...] = keys
    rowmax_key = jnp.max(keys, axis=1, keepdims=True)

    def body(i, t):
        bit = jnp.int32(30) - i
        cand = t | jax.lax.shift_left(jnp.int32(1), bit)
        cnt = jnp.sum((kscratch[...] >= cand).astype(jnp.int32))
        return jnp.where(cnt >= _K, cand, t)

    cntpos = jnp.sum((kscratch[...] >= 0).astype(jnp.int32))
    t0 = jnp.where(cntpos >= _K, jnp.int32(0), jnp.int32(-2147483648))
    tkey = jax.lax.fori_loop(0, 31, body, t0)

    qual = (rowmax_key >= tkey).astype(jnp.float32)
    z = jnp.maximum(jnp.sum(qual), 0.001)
    numer = jnp.sum(qual * jnp.maximum(rowmax, 0.0))
    out_ref[0] = jnp.full((8, 128), numer / z, jnp.float32)


def _loss_kernel(s_ref, out_ref):
    x = s_ref[...]
    diff = x[_B:] - x[:_B]
    sp = jnp.maximum(diff, 0.0) + jnp.log1p(jnp.exp(-jnp.abs(diff)))
    out_ref[...] = jnp.mean(sp, axis=0)


@functools.partial(jax.jit)
def kernel(query_embeddings, doc_embeddings, neg_doc_embeddings):
    docs = jnp.concatenate([doc_embeddings, neg_doc_embeddings], axis=0)

    scores = pl.pallas_call(
        _score_kernel,
        grid=(2 * _B,),
        in_specs=[
            pl.BlockSpec((1, _NQ, _D), lambda i: (i % _B, 0, 0)),
            pl.BlockSpec((1, _NS, _D), lambda i: (i, 0, 0)),
        ],
        out_specs=pl.BlockSpec((1, 8, 128), lambda i: (i, 0, 0)),
        out_shape=jax.ShapeDtypeStruct((2 * _B, 8, 128), jnp.float32),
        scratch_shapes=[pltpu.VMEM((_NQ, _NS), jnp.int32)],
    )(query_embeddings, docs)

    loss = pl.pallas_call(
        _loss_kernel,
        out_shape=jax.ShapeDtypeStruct((8, 128), jnp.float32),
    )(scores)
    return loss[0, 0]


# speedup vs baseline: 17.1086x; 2.2825x over previous
import functools

import jax
import jax.numpy as jnp
from jax.experimental import pallas as pl
from jax.experimental.pallas import tpu as pltpu

_B, _NQ, _NS, _D = 16, 512, 2048, 64
_K = 128


def _count16(ref, cand_i32):
    cand = cand_i32.astype(jnp.int16)
    x = (ref[...] >= cand).astype(jnp.int16)
    r = x.shape[0]
    while r > 1:
        h = r // 2
        x = x[:h] + x[h:]
        r = h
    return jnp.sum(x.astype(jnp.int32))


def _bisect16(ref, need):
    cntpos = _count16(ref, jnp.int32(0))
    t0 = jnp.where(cntpos >= need, jnp.int32(0), jnp.int32(-32768))

    def body(i, t):
        bit = jnp.int32(14) - i
        cand = t | jax.lax.shift_left(jnp.int32(1), bit)
        return jnp.where(_count16(ref, cand) >= need, cand, t)

    return jax.lax.fori_loop(0, 15, body, t0)


def _score_kernel(q_ref, d_ref, out_ref, hi_ref, lo_ref):
    qb = q_ref[0]
    db = d_ref[0]
    s = jax.lax.dot_general(
        qb, db, (((1,), (1,)), ((), ())),
        preferred_element_type=jnp.float32)

    rowmax = jnp.max(s, axis=1, keepdims=True)
    bits = jax.lax.bitcast_convert_type(s, jnp.int32)
    keys = jnp.where(bits < 0, bits ^ jnp.int32(0x7FFFFFFF), bits)
    rowmax_key = jnp.max(keys, axis=1, keepdims=True)

    hi_ref[...] = jax.lax.shift_right_arithmetic(keys, 16).astype(jnp.int16)
    lo_ref[...] = ((keys & jnp.int32(0xFFFF)) ^ jnp.int32(0x8000)).astype(
        jnp.int16)

    hstar = _bisect16(hi_ref, jnp.int32(_K))
    cnt_higher = _count16(hi_ref, hstar + 1)
    need = jnp.int32(_K) - cnt_higher

    hstar16 = hstar.astype(jnp.int16)
    lo_ref[...] = jnp.where(hi_ref[...] == hstar16, lo_ref[...],
                            jnp.int16(-32768))
    lstar = _bisect16(lo_ref, need)

    tkey = jax.lax.shift_left(hstar, 16) | ((lstar ^ jnp.int32(0x8000))
                                            & jnp.int32(0xFFFF))

    qual = (rowmax_key >= tkey).astype(jnp.float32)
    z = jnp.maximum(jnp.sum(qual), 0.001)
    numer = jnp.sum(qual * jnp.maximum(rowmax, 0.0))
    out_ref[0] = jnp.full((8, 128), numer / z, jnp.float32)


def _loss_kernel(s_ref, out_ref):
    x = s_ref[...]
    diff = x[_B:] - x[:_B]
    sp = jnp.maximum(diff, 0.0) + jnp.log1p(jnp.exp(-jnp.abs(diff)))
    out_ref[...] = jnp.mean(sp, axis=0)


@functools.partial(jax.jit)
def kernel(query_embeddings, doc_embeddings, neg_doc_embeddings):
    docs = jnp.concatenate([doc_embeddings, neg_doc_embeddings], axis=0)

    scores = pl.pallas_call(
        _score_kernel,
        grid=(2 * _B,),
        in_specs=[
            pl.BlockSpec((1, _NQ, _D), lambda i: (i % _B, 0, 0)),
            pl.BlockSpec((1, _NS, _D), lambda i: (i, 0, 0)),
        ],
        out_specs=pl.BlockSpec((1, 8, 128), lambda i: (i, 0, 0)),
        out_shape=jax.ShapeDtypeStruct((2 * _B, 8, 128), jnp.float32),
        scratch_shapes=[pltpu.VMEM((_NQ, _NS), jnp.int16),
                        pltpu.VMEM((_NQ, _NS), jnp.int16)],
    )(query_embeddings, docs)

    loss = pl.pallas_call(
        _loss_kernel,
        out_shape=jax.ShapeDtypeStruct((8, 128), jnp.float32),
    )(scores)
    return loss[0, 0]


# speedup vs baseline: 28.1518x; 1.6455x over previous
import functools

import jax
import jax.numpy as jnp
from jax.experimental import pallas as pl
from jax.experimental.pallas import tpu as pltpu

_B, _NQ, _NS, _D = 16, 512, 2048, 64
_K = 128


def _rowmax_kernel(q_ref, d_ref, out_ref):
    qb = q_ref[0]
    db = d_ref[0]
    s2 = jax.lax.dot_general(
        db, qb, (((1,), (1,)), ((), ())),
        preferred_element_type=jnp.float32)
    out_ref[0] = jnp.max(s2, axis=0, keepdims=True)


def _count16(ref, cand_i32):
    cand = cand_i32.astype(jnp.int16)
    x = (ref[...] >= cand).astype(jnp.int16)
    r = x.shape[0]
    while r > 1:
        h = r // 2
        x = x[:h] + x[h:]
        r = h
    return jnp.sum(x.astype(jnp.int32))


def _bisect16(ref, need):
    cntpos = _count16(ref, jnp.int32(0))
    t0 = jnp.where(cntpos >= need, jnp.int32(0), jnp.int32(-32768))

    def body(i, t):
        bit = jnp.int32(14) - i
        cand = t | jax.lax.shift_left(jnp.int32(1), bit)
        return jnp.where(_count16(ref, cand) >= need, cand, t)

    return jax.lax.fori_loop(0, 15, body, t0)


def _score_kernel(q_ref, d_ref, out_ref, hi_ref, lo_ref):
    qb = q_ref[0]
    db = d_ref[0]
    s = jax.lax.dot_general(
        qb, db, (((1,), (1,)), ((), ())),
        preferred_element_type=jnp.float32)

    rowmax = jnp.max(s, axis=1, keepdims=True)
    bits = jax.lax.bitcast_convert_type(s, jnp.int32)
    keys = jnp.where(bits < 0, bits ^ jnp.int32(0x7FFFFFFF), bits)
    rowmax_key = jnp.max(keys, axis=1, keepdims=True)

    hi_ref[...] = jax.lax.shift_right_arithmetic(keys, 16).astype(jnp.int16)
    lo_ref[...] = ((keys & jnp.int32(0xFFFF)) ^ jnp.int32(0x8000)).astype(
        jnp.int16)

    hstar = _bisect16(hi_ref, jnp.int32(_K))
    cnt_higher = _count16(hi_ref, hstar + 1)
    need = jnp.int32(_K) - cnt_higher

    hstar16 = hstar.astype(jnp.int16)
    lo_ref[...] = jnp.where(hi_ref[...] == hstar16, lo_ref[...],
                            jnp.int16(-32768))
    lstar = _bisect16(lo_ref, need)

    tkey = jax.lax.shift_left(hstar, 16) | ((lstar ^ jnp.int32(0x8000))
                                            & jnp.int32(0xFFFF))

    qual = (rowmax_key >= tkey).astype(jnp.float32)
    z = jnp.maximum(jnp.sum(qual), 0.001)
    numer = jnp.sum(qual * jnp.maximum(rowmax, 0.0))
    out_ref[0] = jnp.full((8, 128), numer / z, jnp.float32)


def _loss_kernel(s_ref, out_ref):
    x = s_ref[...]
    diff = x[_B:] - x[:_B]
    sp = jnp.maximum(diff, 0.0) + jnp.log1p(jnp.exp(-jnp.abs(diff)))
    out_ref[...] = jnp.mean(sp, axis=0)


@functools.partial(jax.jit)
def kernel(query_embeddings, doc_embeddings, neg_doc_embeddings):
    docs = jnp.concatenate([doc_embeddings, neg_doc_embeddings], axis=0)

    rowmax = pl.pallas_call(
        _rowmax_kernel,
        grid=(2 * _B,),
        in_specs=[
            pl.BlockSpec((1, _NQ, _D), lambda i: (i % _B, 0, 0)),
            pl.BlockSpec((1, _NS, _D), lambda i: (i, 0, 0)),
        ],
        out_specs=pl.BlockSpec((1, 1, _NQ), lambda i: (i, 0, 0)),
        out_shape=jax.ShapeDtypeStruct((2 * _B, 1, _NQ), jnp.float32),
    )(query_embeddings, docs)

    _, idx = jax.lax.top_k(rowmax[:, 0, :], _K)
    q2 = jnp.concatenate([query_embeddings, query_embeddings], axis=0)
    qc = jnp.take_along_axis(q2, idx[:, :, None], axis=1)

    scores = pl.pallas_call(
        _score_kernel,
        grid=(2 * _B,),
        in_specs=[
            pl.BlockSpec((1, _K, _D), lambda i: (i, 0, 0)),
            pl.BlockSpec((1, _NS, _D), lambda i: (i, 0, 0)),
        ],
        out_specs=pl.BlockSpec((1, 8, 128), lambda i: (i, 0, 0)),
        out_shape=jax.ShapeDtypeStruct((2 * _B, 8, 128), jnp.float32),
        scratch_shapes=[pltpu.VMEM((_K, _NS), jnp.int16),
                        pltpu.VMEM((_K, _NS), jnp.int16)],
    )(qc, docs)

    loss = pl.pallas_call(
        _loss_kernel,
        out_shape=jax.ShapeDtypeStruct((8, 128), jnp.float32),
    )(scores)
    return loss[0, 0]


# speedup vs baseline: 29.9566x; 1.0641x over previous
import functools

import jax
import jax.numpy as jnp
from jax.experimental import pallas as pl
from jax.experimental.pallas import tpu as pltpu

_B, _NQ, _NS, _D = 16, 512, 2048, 64
_K = 128


def _rowmax_kernel(q_ref, d_ref, n_ref, outd_ref, outn_ref):
    qb = q_ref[0]

    def rowmax(db):
        s2 = jax.lax.dot_general(
            db, qb, (((1,), (1,)), ((), ())),
            preferred_element_type=jnp.float32)
        return jnp.max(s2, axis=0, keepdims=True)

    outd_ref[0] = rowmax(d_ref[0])
    outn_ref[0] = rowmax(n_ref[0])


def _count16(ref, cand_i32):
    cand = cand_i32.astype(jnp.int16)
    x = (ref[...] >= cand).astype(jnp.int16)
    r = x.shape[0]
    while r > 1:
        h = r // 2
        x = x[:h] + x[h:]
        r = h
    return jnp.sum(x.astype(jnp.int32))


def _bisect16(ref, need):
    cntpos = _count16(ref, jnp.int32(0))
    t0 = jnp.where(cntpos >= need, jnp.int32(0), jnp.int32(-32768))

    def body(i, t):
        bit = jnp.int32(14) - i
        cand = t | jax.lax.shift_left(jnp.int32(1), bit)
        return jnp.where(_count16(ref, cand) >= need, cand, t)

    return jax.lax.fori_loop(0, 15, body, t0)


def _side_score(qc, db, hi_ref, lo_ref):
    s = jax.lax.dot_general(
        qc, db, (((1,), (1,)), ((), ())),
        preferred_element_type=jnp.float32)

    rowmax = jnp.max(s, axis=1, keepdims=True)
    bits = jax.lax.bitcast_convert_type(s, jnp.int32)
    keys = jnp.where(bits < 0, bits ^ jnp.int32(0x7FFFFFFF), bits)
    rowmax_key = jnp.max(keys, axis=1, keepdims=True)

    hi_ref[...] = jax.lax.shift_right_arithmetic(keys, 16).astype(jnp.int16)
    lo_ref[...] = ((keys & jnp.int32(0xFFFF)) ^ jnp.int32(0x8000)).astype(
        jnp.int16)

    hstar = _bisect16(hi_ref, jnp.int32(_K))
    cnt_higher = _count16(hi_ref, hstar + 1)
    need = jnp.int32(_K) - cnt_higher

    hstar16 = hstar.astype(jnp.int16)
    lo_ref[...] = jnp.where(hi_ref[...] == hstar16, lo_ref[...],
                            jnp.int16(-32768))
    lstar = _bisect16(lo_ref, need)

    tkey = jax.lax.shift_left(hstar, 16) | ((lstar ^ jnp.int32(0x8000))
                                            & jnp.int32(0xFFFF))

    qual = (rowmax_key >= tkey).astype(jnp.float32)
    z = jnp.maximum(jnp.sum(qual), 0.001)
    numer = jnp.sum(qual * jnp.maximum(rowmax, 0.0))
    return numer / z


def _score_kernel(qcd_ref, qcn_ref, d_ref, n_ref, out_ref, hi_ref, lo_ref):
    pos = _side_score(qcd_ref[0], d_ref[0], hi_ref, lo_ref)
    neg = _side_score(qcn_ref[0], n_ref[0], hi_ref, lo_ref)
    diff = neg - pos
    sp = jnp.maximum(diff, 0.0) + jnp.log1p(jnp.exp(-jnp.abs(diff)))

    @pl.when(pl.program_id(0) == 0)
    def _():
        out_ref[...] = jnp.zeros((8, 128), jnp.float32)

    out_ref[...] += sp / _B


@functools.partial(jax.jit)
def kernel(query_embeddings, doc_embeddings, neg_doc_embeddings):
    rmd, rmn = pl.pallas_call(
        _rowmax_kernel,
        grid=(_B,),
        in_specs=[
            pl.BlockSpec((1, _NQ, _D), lambda i: (i, 0, 0)),
            pl.BlockSpec((1, _NS, _D), lambda i: (i, 0, 0)),
            pl.BlockSpec((1, _NS, _D), lambda i: (i, 0, 0)),
        ],
        out_specs=[
            pl.BlockSpec((1, 1, _NQ), lambda i: (i, 0, 0)),
            pl.BlockSpec((1, 1, _NQ), lambda i: (i, 0, 0)),
        ],
        out_shape=[
            jax.ShapeDtypeStruct((_B, 1, _NQ), jnp.float32),
            jax.ShapeDtypeStruct((_B, 1, _NQ), jnp.float32),
        ],
    )(query_embeddings, doc_embeddings, neg_doc_embeddings)

    _, idxd = jax.lax.top_k(rmd[:, 0, :], _K)
    _, idxn = jax.lax.top_k(rmn[:, 0, :], _K)
    qcd = jnp.take_along_axis(query_embeddings, idxd[:, :, None], axis=1)
    qcn = jnp.take_along_axis(query_embeddings, idxn[:, :, None], axis=1)

    loss = pl.pallas_call(
        _score_kernel,
        grid=(_B,),
        in_specs=[
            pl.BlockSpec((1, _K, _D), lambda i: (i, 0, 0)),
            pl.BlockSpec((1, _K, _D), lambda i: (i, 0, 0)),
            pl.BlockSpec((1, _NS, _D), lambda i: (i, 0, 0)),
            pl.BlockSpec((1, _NS, _D), lambda i: (i, 0, 0)),
        ],
        out_specs=pl.BlockSpec((8, 128), lambda i: (0, 0)),
        out_shape=jax.ShapeDtypeStruct((8, 128), jnp.float32),
        scratch_shapes=[pltpu.VMEM((_K, _NS), jnp.int16),
                        pltpu.VMEM((_K, _NS), jnp.int16)],
    )(qcd, qcn, doc_embeddings, neg_doc_embeddings)
    return loss[0, 0]


# speedup vs baseline: 33.5123x; 1.1187x over previous
import functools

import jax
import jax.numpy as jnp
from jax.experimental import pallas as pl
from jax.experimental.pallas import tpu as pltpu

_B, _NQ, _NS, _D = 16, 512, 2048, 64
_K = 128


def _maxes_kernel(q_ref, d_ref, n_ref, rmd_ref, rmn_ref, cmd_ref, cmn_ref):
    qb = q_ref[0]

    def maxes(db, rm_ref, cm_ref):
        s = jax.lax.dot_general(
            db, qb, (((1,), (1,)), ((), ())),
            preferred_element_type=jnp.float32)
        rm_ref[0] = jnp.max(s, axis=0, keepdims=True)
        s2 = jax.lax.dot_general(
            qb, db, (((1,), (1,)), ((), ())),
            preferred_element_type=jnp.float32)
        cm_ref[0] = jnp.max(s2, axis=0, keepdims=True)

    maxes(d_ref[0], rmd_ref, cmd_ref)
    maxes(n_ref[0], rmn_ref, cmn_ref)


def _count16(ref, cand_i32):
    cand = cand_i32.astype(jnp.int16)
    x = (ref[...] >= cand).astype(jnp.int16)
    r = x.shape[0]
    while r > 1:
        h = r // 2
        x = x[:h] + x[h:]
        r = h
    return jnp.sum(x.astype(jnp.int32))


def _bisect16(ref, need):
    cntpos = _count16(ref, jnp.int32(0))
    t0 = jnp.where(cntpos >= need, jnp.int32(0), jnp.int32(-32768))

    def body(i, t):
        bit = jnp.int32(14) - i
        cand = t | jax.lax.shift_left(jnp.int32(1), bit)
        return jnp.where(_count16(ref, cand) >= need, cand, t)

    return jax.lax.fori_loop(0, 15, body, t0)


def _side_score(qc, dc, rv, hi_ref, lo_ref):
    s = jax.lax.dot_general(
        qc, dc, (((1,), (1,)), ((), ())),
        preferred_element_type=jnp.float32)

    bits = jax.lax.bitcast_convert_type(s, jnp.int32)
    keys = jnp.where(bits < 0, bits ^ jnp.int32(0x7FFFFFFF), bits)

    hi_ref[...] = jax.lax.shift_right_arithmetic(keys, 16).astype(jnp.int16)
    lo_ref[...] = ((keys & jnp.int32(0xFFFF)) ^ jnp.int32(0x8000)).astype(
        jnp.int16)

    hstar = _bisect16(hi_ref, jnp.int32(_K))
    cnt_higher = _count16(hi_ref, hstar + 1)
    need = jnp.int32(_K) - cnt_higher

    hstar16 = hstar.astype(jnp.int16)
    lo_ref[...] = jnp.where(hi_ref[...] == hstar16, lo_ref[...],
                            jnp.int16(-32768))
    lstar = _bisect16(lo_ref, need)

    tkey = jax.lax.shift_left(hstar, 16) | ((lstar ^ jnp.int32(0x8000))
                                            & jnp.int32(0xFFFF))

    rbits = jax.lax.bitcast_convert_type(rv, jnp.int32)
    rkeys = jnp.where(rbits < 0, rbits ^ jnp.int32(0x7FFFFFFF), rbits)
    qual = (rkeys >= tkey).astype(jnp.float32)
    z = jnp.maximum(jnp.sum(qual), 0.001)
    numer = jnp.sum(qual * jnp.maximum(rv, 0.0))
    return numer / z


def _score_kernel(qcd_ref, qcn_ref, dcd_ref, dcn_ref, rvd_ref, rvn_ref,
                  out_ref, hi_ref, lo_ref):
    pos = _side_score(qcd_ref[0], dcd_ref[0], rvd_ref[0], hi_ref, lo_ref)
    neg = _side_score(qcn_ref[0], dcn_ref[0], rvn_ref[0], hi_ref, lo_ref)
    diff = neg - pos
    sp = jnp.maximum(diff, 0.0) + jnp.log1p(jnp.exp(-jnp.abs(diff)))

    @pl.when(pl.program_id(0) == 0)
    def _():
        out_ref[...] = jnp.zeros((8, 128), jnp.float32)

    out_ref[...] += sp / _B


@functools.partial(jax.jit)
def kernel(query_embeddings, doc_embeddings, neg_doc_embeddings):
    rmd, rmn, cmd, cmn = pl.pallas_call(
        _maxes_kernel,
        grid=(_B,),
        in_specs=[
            pl.BlockSpec((1, _NQ, _D), lambda i: (i, 0, 0)),
            pl.BlockSpec((1, _NS, _D), lambda i: (i, 0, 0)),
            pl.BlockSpec((1, _NS, _D), lambda i: (i, 0, 0)),
        ],
        out_specs=[
            pl.BlockSpec((1, 1, _NQ), lambda i: (i, 0, 0)),
            pl.BlockSpec((1, 1, _NQ), lambda i: (i, 0, 0)),
            pl.BlockSpec((1, 1, _NS), lambda i: (i, 0, 0)),
            pl.BlockSpec((1, 1, _NS), lambda i: (i, 0, 0)),
        ],
        out_shape=[
            jax.ShapeDtypeStruct((_B, 1, _NQ), jnp.float32),
            jax.ShapeDtypeStruct((_B, 1, _NQ), jnp.float32),
            jax.ShapeDtypeStruct((_B, 1, _NS), jnp.float32),
            jax.ShapeDtypeStruct((_B, 1, _NS), jnp.float32),
        ],
    )(query_embeddings, doc_embeddings, neg_doc_embeddings)

    rvd, idxd = jax.lax.top_k(rmd[:, 0, :], _K)
    rvn, idxn = jax.lax.top_k(rmn[:, 0, :], _K)
    _, cidxd = jax.lax.top_k(cmd[:, 0, :], _K)
    _, cidxn = jax.lax.top_k(cmn[:, 0, :], _K)
    qcd = jnp.take_along_axis(query_embeddings, idxd[:, :, None], axis=1)
    qcn = jnp.take_along_axis(query_embeddings, idxn[:, :, None], axis=1)
    dcd = jnp.take_along_axis(doc_embeddings, cidxd[:, :, None], axis=1)
    dcn = jnp.take_along_axis(neg_doc_embeddings, cidxn[:, :, None], axis=1)

    loss = pl.pallas_call(
        _score_kernel,
        grid=(_B,),
        in_specs=[
            pl.BlockSpec((1, _K, _D), lambda i: (i, 0, 0)),
            pl.BlockSpec((1, _K, _D), lambda i: (i, 0, 0)),
            pl.BlockSpec((1, _K, _D), lambda i: (i, 0, 0)),
            pl.BlockSpec((1, _K, _D), lambda i: (i, 0, 0)),
            pl.BlockSpec((1, 1, _K), lambda i: (i, 0, 0)),
            pl.BlockSpec((1, 1, _K), lambda i: (i, 0, 0)),
        ],
        out_specs=pl.BlockSpec((8, 128), lambda i: (0, 0)),
        out_shape=jax.ShapeDtypeStruct((8, 128), jnp.float32),
        scratch_shapes=[pltpu.VMEM((_K, _K), jnp.int16),
                        pltpu.VMEM((_K, _K), jnp.int16)],
    )(qcd, qcn, dcd, dcn, rvd[:, None, :], rvn[:, None, :])
    return loss[0, 0]


# speedup vs baseline: 43.9104x; 1.3103x over previous
import functools

import jax
import jax.numpy as jnp
from jax.experimental import pallas as pl
from jax.experimental.pallas import tpu as pltpu

_B, _NQ, _NS, _D = 16, 512, 2048, 64
_K = 128


def _maxes_kernel(q_ref, d_ref, n_ref, rmd_ref, rmn_ref, cmd_ref, cmn_ref):
    qb = q_ref[0]

    def maxes(db, rm_ref, cm_ref):
        s = jax.lax.dot_general(
            db, qb, (((1,), (1,)), ((), ())),
            preferred_element_type=jnp.float32)
        rm_ref[0] = jnp.max(s, axis=0, keepdims=True)
        s2 = jax.lax.dot_general(
            qb, db, (((1,), (1,)), ((), ())),
            preferred_element_type=jnp.float32)
        cm_ref[0] = jnp.max(s2, axis=0, keepdims=True)

    maxes(d_ref[0], rmd_ref, cmd_ref)
    maxes(n_ref[0], rmn_ref, cmn_ref)


def _count16(ref, cand_i32):
    cand = cand_i32.astype(jnp.int16)
    x = (ref[...] >= cand).astype(jnp.int16)
    r = x.shape[0]
    while r > 1:
        h = r // 2
        x = x[:h] + x[h:]
        r = h
    return jnp.sum(x.astype(jnp.int32))


def _bisect16_pair(ref_a, ref_b, need_a, need_b):
    ta = jnp.where(_count16(ref_a, jnp.int32(0)) >= need_a,
                   jnp.int32(0), jnp.int32(-32768))
    tb = jnp.where(_count16(ref_b, jnp.int32(0)) >= need_b,
                   jnp.int32(0), jnp.int32(-32768))

    def body(i, ts):
        ta, tb = ts
        bit = jax.lax.shift_left(jnp.int32(1), jnp.int32(14) - i)
        ca = ta | bit
        cb = tb | bit
        ta = jnp.where(_count16(ref_a, ca) >= need_a, ca, ta)
        tb = jnp.where(_count16(ref_b, cb) >= need_b, cb, tb)
        return ta, tb

    return jax.lax.fori_loop(0, 15, body, (ta, tb))


def _side_keys(qc, dc, hi_ref, lo_ref):
    s = jax.lax.dot_general(
        qc, dc, (((1,), (1,)), ((), ())),
        preferred_element_type=jnp.float32)

    bits = jax.lax.bitcast_convert_type(s, jnp.int32)
    keys = jnp.where(bits < 0, bits ^ jnp.int32(0x7FFFFFFF), bits)
    hi_ref[...] = jax.lax.shift_right_arithmetic(keys, 16).astype(jnp.int16)
    lo_ref[...] = ((keys & jnp.int32(0xFFFF)) ^ jnp.int32(0x8000)).astype(
        jnp.int16)


def _qual_score(rv, tkey):
    rbits = jax.lax.bitcast_convert_type(rv, jnp.int32)
    rkeys = jnp.where(rbits < 0, rbits ^ jnp.int32(0x7FFFFFFF), rbits)
    qual = (rkeys >= tkey).astype(jnp.float32)
    z = jnp.maximum(jnp.sum(qual), 0.001)
    numer = jnp.sum(qual * jnp.maximum(rv, 0.0))
    return numer / z


def _score_kernel(qcd_ref, qcn_ref, dcd_ref, dcn_ref, rvd_ref, rvn_ref,
                  out_ref, hip_ref, lop_ref, hin_ref, lon_ref):
    _side_keys(qcd_ref[0], dcd_ref[0], hip_ref, lop_ref)
    _side_keys(qcn_ref[0], dcn_ref[0], hin_ref, lon_ref)

    kk = jnp.int32(_K)
    hp, hn = _bisect16_pair(hip_ref, hin_ref, kk, kk)
    need_p = kk - _count16(hip_ref, hp + 1)
    need_n = kk - _count16(hin_ref, hn + 1)

    lop_ref[...] = jnp.where(hip_ref[...] == hp.astype(jnp.int16),
                             lop_ref[...], jnp.int16(-32768))
    lon_ref[...] = jnp.where(hin_ref[...] == hn.astype(jnp.int16),
                             lon_ref[...], jnp.int16(-32768))
    lp, ln = _bisect16_pair(lop_ref, lon_ref, need_p, need_n)

    tkey_p = jax.lax.shift_left(hp, 16) | ((lp ^ jnp.int32(0x8000))
                                           & jnp.int32(0xFFFF))
    tkey_n = jax.lax.shift_left(hn, 16) | ((ln ^ jnp.int32(0x8000))
                                           & jnp.int32(0xFFFF))

    pos = _qual_score(rvd_ref[0], tkey_p)
    neg = _qual_score(rvn_ref[0], tkey_n)
    diff = neg - pos
    sp = jnp.maximum(diff, 0.0) + jnp.log1p(jnp.exp(-jnp.abs(diff)))

    @pl.when(pl.program_id(0) == 0)
    def _():
        out_ref[...] = jnp.zeros((8, 128), jnp.float32)

    out_ref[...] += sp / _B


@functools.partial(jax.jit)
def kernel(query_embeddings, doc_embeddings, neg_doc_embeddings):
    rmd, rmn, cmd, cmn = pl.pallas_call(
        _maxes_kernel,
        grid=(_B,),
        in_specs=[
            pl.BlockSpec((1, _NQ, _D), lambda i: (i, 0, 0)),
            pl.BlockSpec((1, _NS, _D), lambda i: (i, 0, 0)),
            pl.BlockSpec((1, _NS, _D), lambda i: (i, 0, 0)),
        ],
        out_specs=[
            pl.BlockSpec((1, 1, _NQ), lambda i: (i, 0, 0)),
            pl.BlockSpec((1, 1, _NQ), lambda i: (i, 0, 0)),
            pl.BlockSpec((1, 1, _NS), lambda i: (i, 0, 0)),
            pl.BlockSpec((1, 1, _NS), lambda i: (i, 0, 0)),
        ],
        out_shape=[
            jax.ShapeDtypeStruct((_B, 1, _NQ), jnp.float32),
            jax.ShapeDtypeStruct((_B, 1, _NQ), jnp.float32),
            jax.ShapeDtypeStruct((_B, 1, _NS), jnp.float32),
            jax.ShapeDtypeStruct((_B, 1, _NS), jnp.float32),
        ],
    )(query_embeddings, doc_embeddings, neg_doc_embeddings)

    rvd, idxd = jax.lax.top_k(rmd[:, 0, :], _K)
    rvn, idxn = jax.lax.top_k(rmn[:, 0, :], _K)
    _, cidxd = jax.lax.top_k(cmd[:, 0, :], _K)
    _, cidxn = jax.lax.top_k(cmn[:, 0, :], _K)
    qcd = jnp.take_along_axis(query_embeddings, idxd[:, :, None], axis=1)
    qcn = jnp.take_along_axis(query_embeddings, idxn[:, :, None], axis=1)
    dcd = jnp.take_along_axis(doc_embeddings, cidxd[:, :, None], axis=1)
    dcn = jnp.take_along_axis(neg_doc_embeddings, cidxn[:, :, None], axis=1)

    loss = pl.pallas_call(
        _score_kernel,
        grid=(_B,),
        in_specs=[
            pl.BlockSpec((1, _K, _D), lambda i: (i, 0, 0)),
            pl.BlockSpec((1, _K, _D), lambda i: (i, 0, 0)),
            pl.BlockSpec((1, _K, _D), lambda i: (i, 0, 0)),
            pl.BlockSpec((1, _K, _D), lambda i: (i, 0, 0)),
            pl.BlockSpec((1, 1, _K), lambda i: (i, 0, 0)),
            pl.BlockSpec((1, 1, _K), lambda i: (i, 0, 0)),
        ],
        out_specs=pl.BlockSpec((8, 128), lambda i: (0, 0)),
        out_shape=jax.ShapeDtypeStruct((8, 128), jnp.float32),
        scratch_shapes=[pltpu.VMEM((_K, _K), jnp.int16),
                        pltpu.VMEM((_K, _K), jnp.int16),
                        pltpu.VMEM((_K, _K), jnp.int16),
                        pltpu.VMEM((_K, _K), jnp.int16)],
    )(qcd, qcn, dcd, dcn, rvd[:, None, :], rvn[:, None, :])
    return loss[0, 0]
